# Initial kernel scaffold; baseline (speedup 1.0000x reference)
import jax
import jax.numpy as jnp
from jax.experimental import pallas as pl


def kernel(idx, table):
    raise NotImplementedError("write your pallas kernel here")


# speedup vs baseline: 1.0141x; 1.0141x over previous
import functools

import jax
import jax.numpy as jnp
from jax import lax
from jax.experimental import pallas as pl
from jax.experimental.pallas import tpu as pltpu
from jax.experimental.pallas import tpu_sc as plsc

VOCAB = 1000
NUM_CORES = 2
NUM_SUBCORES = 16
NW = NUM_CORES * NUM_SUBCORES
TOTAL = 1024 * 50
BPW = TOTAL // NW
CHUNK = 64
NCHUNK = BPW // CHUNK

_MESH = plsc.VectorSubcoreMesh(core_axis_name="c", subcore_axis_name="s")


@functools.partial(
    pl.kernel,
    out_type=jax.ShapeDtypeStruct((TOTAL, VOCAB), jnp.float32),
    mesh=_MESH,
    compiler_params=pltpu.CompilerParams(use_tc_tiling_on_sc=False),
    scratch_types=[
        pltpu.VMEM((NCHUNK, CHUNK), jnp.int32),
        pltpu.VMEM((CHUNK, VOCAB), jnp.float32),
        pltpu.SemaphoreType.DMA,
    ],
)
def _gather_kernel(idx_hbm, table_hbm, out_hbm, idx_v, rows_v, gsem):
    wid = lax.axis_index("s") * NUM_CORES + lax.axis_index("c")
    base = wid * BPW
    pltpu.sync_copy(idx_hbm.at[wid], idx_v)

    def body(i, carry):
        pltpu.async_copy(table_hbm.at[idx_v.at[i]], rows_v, gsem).wait()
        start = pl.multiple_of(base + i * CHUNK, CHUNK)
        pltpu.sync_copy(rows_v, out_hbm.at[pl.ds(start, CHUNK)])
        return carry

    lax.fori_loop(0, NCHUNK, body, 0)


def kernel(idx, table):
    flat_idx = idx.reshape(NW, NCHUNK, CHUNK).astype(jnp.int32)
    out = _gather_kernel(flat_idx, table)
    return out.reshape(idx.shape[0], idx.shape[1], VOCAB)


# speedup vs baseline: 1.0282x; 1.0139x over previous
import functools

import jax
import jax.numpy as jnp
from jax import lax
from jax.experimental import pallas as pl
from jax.experimental.pallas import tpu as pltpu
from jax.experimental.pallas import tpu_sc as plsc

VOCAB = 1000
NUM_CORES = 2
NUM_SUBCORES = 16
NW = NUM_CORES * NUM_SUBCORES
TOTAL = 1024 * 50
BPW = TOTAL // NW
CHUNK = 50
NCHUNK = BPW // CHUNK

_MESH = plsc.VectorSubcoreMesh(core_axis_name="c", subcore_axis_name="s")


@functools.partial(
    pl.kernel,
    out_type=jax.ShapeDtypeStruct((TOTAL, VOCAB), jnp.float32),
    mesh=_MESH,
    compiler_params=pltpu.CompilerParams(use_tc_tiling_on_sc=False),
    scratch_types=[
        pltpu.VMEM((NCHUNK, CHUNK), jnp.int32),
        pltpu.VMEM((2, CHUNK, VOCAB), jnp.float32),
        pltpu.SemaphoreType.DMA,
        pltpu.SemaphoreType.DMA,
    ],
)
def _gather_kernel(idx_hbm, table_hbm, out_hbm, idx_v, rows_v, sem0, sem1):
    wid = lax.axis_index("s") * NUM_CORES + lax.axis_index("c")
    base = wid * BPW
    sems = (sem0, sem1)
    pltpu.sync_copy(idx_hbm.at[wid], idx_v)

    pltpu.async_copy(table_hbm.at[idx_v.at[0]], rows_v.at[0], sem0)

    def outer(i2, carry):
        for b in range(2):
            i = i2 * 2 + b
            pltpu.make_async_copy(
                table_hbm.at[idx_v.at[i]], rows_v.at[b], sems[b]
            ).wait()
            if b == 0:
                pltpu.async_copy(
                    table_hbm.at[idx_v.at[i + 1]], rows_v.at[1], sems[1]
                )
            else:

                @pl.when(i2 < NCHUNK // 2 - 1)
                def _():
                    pltpu.async_copy(
                        table_hbm.at[idx_v.at[i + 1]], rows_v.at[0], sems[0]
                    )

            start = pl.multiple_of(base + i * CHUNK, CHUNK)
            pltpu.sync_copy(rows_v.at[b], out_hbm.at[pl.ds(start, CHUNK)])
        return carry

    lax.fori_loop(0, NCHUNK // 2, outer, 0)


def kernel(idx, table):
    flat_idx = idx.reshape(NW, NCHUNK, CHUNK).astype(jnp.int32)
    out = _gather_kernel(flat_idx, table)
    return out.reshape(idx.shape[0], idx.shape[1], VOCAB)


# speedup vs baseline: 1.0287x; 1.0005x over previous
import functools

import jax
import jax.numpy as jnp
from jax import lax
from jax.experimental import pallas as pl
from jax.experimental.pallas import tpu as pltpu
from jax.experimental.pallas import tpu_sc as plsc

VOCAB = 1000
BATCH = 1024
SEQ = 50
NUM_CORES = 2
NUM_SUBCORES = 16
NW = NUM_CORES * NUM_SUBCORES
TOTAL = BATCH * SEQ
BPW = TOTAL // NW
CHUNK = SEQ
NCHUNK = BPW // CHUNK

_MESH = plsc.VectorSubcoreMesh(core_axis_name="c", subcore_axis_name="s")


@functools.partial(
    pl.kernel,
    out_type=jax.ShapeDtypeStruct((BATCH, SEQ, VOCAB), jnp.float32),
    mesh=_MESH,
    compiler_params=pltpu.CompilerParams(use_tc_tiling_on_sc=False),
    scratch_types=[
        pltpu.VMEM((NCHUNK, CHUNK), jnp.int32),
        pltpu.VMEM((2, CHUNK, VOCAB), jnp.float32),
        pltpu.SemaphoreType.DMA,
        pltpu.SemaphoreType.DMA,
    ],
)
def _gather_kernel(idx_hbm, table_hbm, out_hbm, idx_v, rows_v, sem0, sem1):
    wid = lax.axis_index("s") * NUM_CORES + lax.axis_index("c")
    bbase = wid * NCHUNK
    sems = (sem0, sem1)
    pltpu.sync_copy(idx_hbm.at[wid], idx_v)

    pltpu.async_copy(table_hbm.at[idx_v.at[0]], rows_v.at[0], sem0)

    def outer(i2, carry):
        for b in range(2):
            i = i2 * 2 + b
            pltpu.make_async_copy(
                table_hbm.at[idx_v.at[i]], rows_v.at[b], sems[b]
            ).wait()
            if b == 0:
                pltpu.async_copy(
                    table_hbm.at[idx_v.at[i + 1]], rows_v.at[1], sems[1]
                )
            else:

                @pl.when(i2 < NCHUNK // 2 - 1)
                def _():
                    pltpu.async_copy(
                        table_hbm.at[idx_v.at[i + 1]], rows_v.at[0], sems[0]
                    )

            pltpu.sync_copy(rows_v.at[b], out_hbm.at[bbase + i])
        return carry

    lax.fori_loop(0, NCHUNK // 2, outer, 0)


def kernel(idx, table):
    flat_idx = idx.reshape(NW, NCHUNK, CHUNK).astype(jnp.int32)
    return _gather_kernel(flat_idx, table)
